# baseline (speedup 1.0000x reference)
import functools

import jax
import jax.numpy as jnp
from jax import lax
from jax.experimental import pallas as pl
from jax.experimental.pallas import tpu as pltpu
from jax.experimental.pallas import tpu_sc as plsc

S = 2048
D = 768
FFN = 3072
E = 8
CAP = 256
CHUNK = 256
LANES = 16
NW = 32
ROWS_PER_W = S // NW


def _router_body(x_ref, wr_ref, logits_ref, src_ref, rp_ref):
    xv = x_ref[...]
    wr = wr_ref[...]
    logits = lax.dot_general(
        xv, wr, (((1,), (1,)), ((), ())),
        preferred_element_type=jnp.float32)
    logits_ref[...] = logits

    mx = jnp.max(logits, axis=1, keepdims=True)
    ex = jnp.exp(logits - mx)
    sm = jnp.sum(ex, axis=1, keepdims=True)
    probs = ex / sm
    rp = jnp.max(probs, axis=1, keepdims=True)
    rp_ref[...] = rp

    lane = lax.broadcasted_iota(jnp.int32, (S, E), 1)
    am = jnp.min(jnp.where(probs == rp, lane, E), axis=1, keepdims=True)
    oh = (lane == am).astype(jnp.float32)

    r_i = lax.broadcasted_iota(jnp.int32, (CHUNK, CHUNK), 0)
    c_i = lax.broadcasted_iota(jnp.int32, (CHUNK, CHUNK), 1)
    tri = (r_i >= c_i).astype(jnp.float32)
    carry = jnp.zeros((1, E), jnp.float32)
    psel_parts = []
    for k in range(S // CHUNK):
        blk = oh[k * CHUNK:(k + 1) * CHUNK, :]
        pr = lax.dot_general(
            tri, blk, (((1,), (0,)), ((), ())),
            preferred_element_type=jnp.float32) + carry
        carry = pr[CHUNK - 1:CHUNK, :]
        psel_parts.append(jnp.sum(pr * blk, axis=1, keepdims=True))
    psel = jnp.concatenate(psel_parts, axis=0)

    tok = lax.broadcasted_iota(jnp.int32, (S, 1), 0)
    routed = psel <= jnp.float32(CAP)
    slot = am * CAP + psel.astype(jnp.int32) - 1
    src_ref[...] = jnp.where(routed, S + slot, tok)


_router_call = pl.pallas_call(
    _router_body,
    out_shape=(
        jax.ShapeDtypeStruct((S, E), jnp.float32),
        jax.ShapeDtypeStruct((S, 1), jnp.int32),
        jax.ShapeDtypeStruct((S, 1), jnp.float32),
    ),
)


def _sc_dispatch_body(src_hbm, rp_hbm, x_hbm, xg_hbm, rps_hbm,
                      src_v, rp_v, idx_v, idxc_v, rpsc_v, rows_v,
                      sem1, sem2, sem3):
    wid = lax.axis_index("s") * 2 + lax.axis_index("c")
    base = wid * ROWS_PER_W

    cp_src = pltpu.async_copy(src_hbm, src_v, sem1)
    cp_rp = pltpu.async_copy(rp_hbm, rp_v, sem2)

    def init_body(i, _):
        idx_v[pl.ds(base + i * LANES, LANES)] = jnp.zeros((LANES,), jnp.int32)
        return _

    lax.fori_loop(0, ROWS_PER_W // LANES, init_body, 0, unroll=4)

    cp_src.wait()

    def scatter_body(i, _):
        v = src_v[pl.ds(i * LANES, LANES)]
        slot = v - S
        m = slot >= 0
        toks = lax.iota(jnp.int32, LANES) + i * LANES
        plsc.store_scatter(idx_v, [slot], toks, mask=m)
        return _

    lax.fori_loop(0, S // LANES, scatter_body, 0, unroll=4)

    def copy_body(i, _):
        idxc_v[pl.ds(i * LANES, LANES)] = idx_v[pl.ds(base + i * LANES, LANES)]
        return _

    lax.fori_loop(0, ROWS_PER_W // LANES, copy_body, 0, unroll=4)

    gather = pltpu.async_copy(x_hbm.at[idxc_v], rows_v, sem3)
    cp_rp.wait()

    def rps_body(i, _):
        iv = idxc_v[pl.ds(i * LANES, LANES)]
        rpsc_v[pl.ds(i * LANES, LANES)] = plsc.load_gather(rp_v, [iv])
        return _

    lax.fori_loop(0, ROWS_PER_W // LANES, rps_body, 0, unroll=4)

    cp_rps = pltpu.async_copy(rpsc_v, rps_hbm.at[pl.ds(base, ROWS_PER_W)], sem1)
    gather.wait()
    pltpu.sync_copy(rows_v, xg_hbm.at[pl.ds(base, ROWS_PER_W)])
    cp_rps.wait()


@functools.cache
def _sc_dispatch_call():
    return functools.partial(
        pl.kernel,
        mesh=plsc.VectorSubcoreMesh(core_axis_name="c", subcore_axis_name="s"),
        compiler_params=pltpu.CompilerParams(needs_layout_passes=False),
        out_type=(
            jax.ShapeDtypeStruct((S, D), jnp.float32),
            jax.ShapeDtypeStruct((S,), jnp.float32),
        ),
        scratch_types=[
            pltpu.VMEM((S,), jnp.int32),
            pltpu.VMEM((S,), jnp.float32),
            pltpu.VMEM((S,), jnp.int32),
            pltpu.VMEM((ROWS_PER_W,), jnp.int32),
            pltpu.VMEM((ROWS_PER_W,), jnp.float32),
            pltpu.VMEM((ROWS_PER_W, D), jnp.float32),
            pltpu.SemaphoreType.DMA,
            pltpu.SemaphoreType.DMA,
            pltpu.SemaphoreType.DMA,
        ],
    )(_sc_dispatch_body)


FSPLIT = 2
FCHUNK = FFN // FSPLIT


def _expert_body(xg_ref, x_ref, src_ref, rp_ref, rps_ref, wi_ref, wo_ref,
                 y_ref, acc_ref):
    i = pl.program_id(0)
    f = pl.program_id(1)

    @pl.when((i == 0) & (f == 0))
    def _base():
        routed = src_ref[...] >= S
        y_ref[...] = jnp.where(routed, 0.0, rp_ref[...] * x_ref[...])

    @pl.when(i > 0)
    def _expert():
        xg = xg_ref[...]
        wi = wi_ref[0]
        wo = wo_ref[0]
        h = lax.dot_general(
            xg, wi, (((1,), (1,)), ((), ())),
            preferred_element_type=jnp.float32)
        h = 0.5 * h * (1.0 + lax.erf(h * 0.7071067811865476))
        part = lax.dot_general(
            h, wo, (((1,), (1,)), ((), ())),
            preferred_element_type=jnp.float32)

        @pl.when(f == 0)
        def _first():
            acc_ref[...] = part

        @pl.when(f == FSPLIT - 1)
        def _last():
            out = (acc_ref[...] + part) * rps_ref[...]
            c = lax.broadcasted_iota(jnp.int32, (S, CAP), 1)
            de = (src_ref[...] == (S + (i - 1) * CAP + c)).astype(jnp.float32)
            y_ref[...] += lax.dot_general(
                de, out, (((1,), (0,)), ((), ())),
                preferred_element_type=jnp.float32)


_expert_call = pl.pallas_call(
    _expert_body,
    grid=(E + 1, FSPLIT),
    in_specs=[
        pl.BlockSpec((CAP, D), lambda i, f: (jnp.maximum(i - 1, 0), 0)),
        pl.BlockSpec((S, D), lambda i, f: (0, 0)),
        pl.BlockSpec((S, 1), lambda i, f: (0, 0)),
        pl.BlockSpec((S, 1), lambda i, f: (0, 0)),
        pl.BlockSpec((CAP, 1), lambda i, f: (jnp.maximum(i - 1, 0), 0)),
        pl.BlockSpec((1, FCHUNK, D),
                     lambda i, f: (jnp.maximum(i - 1, 0),
                                   jnp.where(i % 2 == 1, FSPLIT - 1 - f, f), 0)),
        pl.BlockSpec((1, D, FCHUNK),
                     lambda i, f: (jnp.maximum(i - 1, 0), 0,
                                   jnp.where(i % 2 == 1, FSPLIT - 1 - f, f))),
    ],
    out_specs=pl.BlockSpec((S, D), lambda i, f: (0, 0)),
    out_shape=jax.ShapeDtypeStruct((S, D), jnp.float32),
    scratch_shapes=[pltpu.VMEM((CAP, D), jnp.float32)],
)


def _sc_combine_body(src_hbm, tab_hbm, y_hbm, idx_v, rows_v, sem):
    wid = lax.axis_index("s") * 2 + lax.axis_index("c")
    base = wid * ROWS_PER_W
    pltpu.sync_copy(src_hbm.at[pl.ds(base, ROWS_PER_W)], idx_v)
    pltpu.async_copy(tab_hbm.at[idx_v], rows_v, sem).wait()
    pltpu.sync_copy(rows_v, y_hbm.at[pl.ds(base, ROWS_PER_W)])


@functools.cache
def _sc_combine_call():
    return functools.partial(
        pl.kernel,
        mesh=plsc.VectorSubcoreMesh(core_axis_name="c", subcore_axis_name="s"),
        compiler_params=pltpu.CompilerParams(needs_layout_passes=False),
        out_type=jax.ShapeDtypeStruct((S, D), jnp.float32),
        scratch_types=[
            pltpu.VMEM((ROWS_PER_W,), jnp.int32),
            pltpu.VMEM((ROWS_PER_W, D), jnp.float32),
            pltpu.SemaphoreType.DMA,
        ],
    )(_sc_combine_body)


def kernel(x, Wr, Wi, Wo):
    x2 = x.reshape(S, D)
    logits, src, rp = _router_call(x2, Wr)
    xg, rps = _sc_dispatch_call()(src.reshape(S), rp.reshape(S), x2)
    y2 = _expert_call(xg, x2, src, rp, rps.reshape(S, 1), Wi, Wo)
    return y2.reshape(1, S, D), logits.reshape(1, S, E)


# speedup vs baseline: 1.0695x; 1.0695x over previous
import functools

import jax
import jax.numpy as jnp
from jax import lax
from jax.experimental import pallas as pl
from jax.experimental.pallas import tpu as pltpu
from jax.experimental.pallas import tpu_sc as plsc

S = 2048
D = 768
FFN = 3072
E = 8
CAP = 256
CHUNK = 256
LANES = 16
NW = 32
ROWS_PER_W = S // NW


def _router_body(x_ref, wr_ref, logits_ref, src_ref, rp_ref):
    xv = x_ref[...]
    wr = wr_ref[...]
    logits = lax.dot_general(
        xv, wr, (((1,), (1,)), ((), ())),
        preferred_element_type=jnp.float32)
    logits_ref[...] = logits

    mx = jnp.max(logits, axis=1, keepdims=True)
    ex = jnp.exp(logits - mx)
    sm = jnp.sum(ex, axis=1, keepdims=True)
    probs = ex / sm
    rp = jnp.max(probs, axis=1, keepdims=True)
    rp_ref[...] = rp

    lane = lax.broadcasted_iota(jnp.int32, (S, E), 1)
    am = jnp.min(jnp.where(probs == rp, lane, E), axis=1, keepdims=True)
    oh = (lane == am).astype(jnp.float32)

    r_i = lax.broadcasted_iota(jnp.int32, (CHUNK, CHUNK), 0)
    c_i = lax.broadcasted_iota(jnp.int32, (CHUNK, CHUNK), 1)
    tri = (r_i >= c_i).astype(jnp.float32)
    carry = jnp.zeros((1, E), jnp.float32)
    psel_parts = []
    for k in range(S // CHUNK):
        blk = oh[k * CHUNK:(k + 1) * CHUNK, :]
        pr = lax.dot_general(
            tri, blk, (((1,), (0,)), ((), ())),
            preferred_element_type=jnp.float32) + carry
        carry = pr[CHUNK - 1:CHUNK, :]
        psel_parts.append(jnp.sum(pr * blk, axis=1, keepdims=True))
    psel = jnp.concatenate(psel_parts, axis=0)

    tok = lax.broadcasted_iota(jnp.int32, (S, 1), 0)
    routed = psel <= jnp.float32(CAP)
    slot = am * CAP + psel.astype(jnp.int32) - 1
    src_ref[...] = jnp.where(routed, S + slot, tok)


_router_call = pl.pallas_call(
    _router_body,
    out_shape=(
        jax.ShapeDtypeStruct((S, E), jnp.float32),
        jax.ShapeDtypeStruct((S, 1), jnp.int32),
        jax.ShapeDtypeStruct((S, 1), jnp.float32),
    ),
)


def _sc_dispatch_body(src_hbm, rp_hbm, x_hbm, xg_hbm, rps_hbm,
                      src_v, rp_v, idx_v, idxc_v, rpsc_v, rows_v,
                      sem1, sem2, sem3):
    wid = lax.axis_index("s") * 2 + lax.axis_index("c")
    base = wid * ROWS_PER_W

    cp_src = pltpu.async_copy(src_hbm, src_v, sem1)
    cp_rp = pltpu.async_copy(rp_hbm, rp_v, sem2)

    def init_body(i, _):
        idx_v[pl.ds(base + i * LANES, LANES)] = jnp.zeros((LANES,), jnp.int32)
        return _

    lax.fori_loop(0, ROWS_PER_W // LANES, init_body, 0, unroll=4)

    cp_src.wait()

    def scatter_body(i, _):
        v = src_v[pl.ds(i * LANES, LANES)]
        slot = v - S
        m = slot >= 0
        toks = lax.iota(jnp.int32, LANES) + i * LANES
        plsc.store_scatter(idx_v, [slot], toks, mask=m)
        return _

    lax.fori_loop(0, S // LANES, scatter_body, 0, unroll=4)

    def copy_body(i, _):
        idxc_v[pl.ds(i * LANES, LANES)] = idx_v[pl.ds(base + i * LANES, LANES)]
        return _

    lax.fori_loop(0, ROWS_PER_W // LANES, copy_body, 0, unroll=4)

    gather = pltpu.async_copy(x_hbm.at[idxc_v], rows_v, sem3)
    cp_rp.wait()

    def rps_body(i, _):
        iv = idxc_v[pl.ds(i * LANES, LANES)]
        rpsc_v[pl.ds(i * LANES, LANES)] = plsc.load_gather(rp_v, [iv])
        return _

    lax.fori_loop(0, ROWS_PER_W // LANES, rps_body, 0, unroll=4)

    cp_rps = pltpu.async_copy(rpsc_v, rps_hbm.at[pl.ds(base, ROWS_PER_W)], sem1)
    gather.wait()
    pltpu.sync_copy(rows_v, xg_hbm.at[pl.ds(base, ROWS_PER_W)])
    cp_rps.wait()


@functools.cache
def _sc_dispatch_call():
    return functools.partial(
        pl.kernel,
        mesh=plsc.VectorSubcoreMesh(core_axis_name="c", subcore_axis_name="s"),
        compiler_params=pltpu.CompilerParams(needs_layout_passes=False),
        out_type=(
            jax.ShapeDtypeStruct((S, D), jnp.float32),
            jax.ShapeDtypeStruct((S,), jnp.float32),
        ),
        scratch_types=[
            pltpu.VMEM((S,), jnp.int32),
            pltpu.VMEM((S,), jnp.float32),
            pltpu.VMEM((S,), jnp.int32),
            pltpu.VMEM((ROWS_PER_W,), jnp.int32),
            pltpu.VMEM((ROWS_PER_W,), jnp.float32),
            pltpu.VMEM((ROWS_PER_W, D), jnp.float32),
            pltpu.SemaphoreType.DMA,
            pltpu.SemaphoreType.DMA,
            pltpu.SemaphoreType.DMA,
        ],
    )(_sc_dispatch_body)


def _expert_body(xg_ref, x_ref, src_ref, rp_ref, rps_ref, wi_ref, wo_ref,
                 y_ref):
    i = pl.program_id(0)

    @pl.when(i == 0)
    def _base():
        routed = src_ref[...] >= S
        y_ref[...] = jnp.where(routed, 0.0, rp_ref[...] * x_ref[...])

    @pl.when(i > 0)
    def _expert():
        xg = xg_ref[...]
        wi = wi_ref[0]
        wo = wo_ref[0]
        h = lax.dot_general(
            xg, wi, (((1,), (1,)), ((), ())),
            preferred_element_type=jnp.float32)
        h = 0.5 * h * (1.0 + lax.erf(h * 0.7071067811865476))
        out = lax.dot_general(
            h, wo, (((1,), (1,)), ((), ())),
            preferred_element_type=jnp.float32)
        out = out * rps_ref[...]
        c = lax.broadcasted_iota(jnp.int32, (S, CAP), 1)
        de = (src_ref[...] == (S + (i - 1) * CAP + c)).astype(jnp.float32)
        y_ref[...] += lax.dot_general(
            de, out, (((1,), (0,)), ((), ())),
            preferred_element_type=jnp.float32)


_expert_call = pl.pallas_call(
    _expert_body,
    grid=(E + 1,),
    in_specs=[
        pl.BlockSpec((CAP, D), lambda i: (jnp.maximum(i - 1, 0), 0)),
        pl.BlockSpec((S, D), lambda i: (0, 0)),
        pl.BlockSpec((S, 1), lambda i: (0, 0)),
        pl.BlockSpec((S, 1), lambda i: (0, 0)),
        pl.BlockSpec((CAP, 1), lambda i: (jnp.maximum(i - 1, 0), 0)),
        pl.BlockSpec((1, FFN, D), lambda i: (jnp.maximum(i - 1, 0), 0, 0)),
        pl.BlockSpec((1, D, FFN), lambda i: (jnp.maximum(i - 1, 0), 0, 0)),
    ],
    out_specs=pl.BlockSpec((S, D), lambda i: (0, 0)),
    out_shape=jax.ShapeDtypeStruct((S, D), jnp.float32),
)


def kernel(x, Wr, Wi, Wo):
    x2 = x.reshape(S, D)
    logits, src, rp = _router_call(x2, Wr)
    xg, rps = _sc_dispatch_call()(src.reshape(S), rp.reshape(S), x2)
    y2 = _expert_call(xg, x2, src, rp, rps.reshape(S, 1), Wi, Wo)
    return y2.reshape(1, S, D), logits.reshape(1, S, E)


# speedup vs baseline: 1.1161x; 1.0435x over previous
import functools

import jax
import jax.numpy as jnp
from jax import lax
from jax.experimental import pallas as pl
from jax.experimental.pallas import tpu as pltpu
from jax.experimental.pallas import tpu_sc as plsc

S = 2048
D = 768
FFN = 3072
E = 8
CAP = 256
CHUNK = 256
LANES = 16
NW = 32
ROWS_PER_W = S // NW


def _router_body(x_ref, wr_ref, logits_ref, src_ref, rp_ref):
    xv = x_ref[...]
    wr = wr_ref[...]
    logits = lax.dot_general(
        xv, wr, (((1,), (1,)), ((), ())),
        preferred_element_type=jnp.float32)
    logits_ref[...] = logits

    mx = jnp.max(logits, axis=1, keepdims=True)
    ex = jnp.exp(logits - mx)
    sm = jnp.sum(ex, axis=1, keepdims=True)
    probs = ex / sm
    rp = jnp.max(probs, axis=1, keepdims=True)
    rp_ref[...] = rp

    lane = lax.broadcasted_iota(jnp.int32, (S, E), 1)
    am = jnp.min(jnp.where(probs == rp, lane, E), axis=1, keepdims=True)
    oh = (lane == am).astype(jnp.float32)

    r_i = lax.broadcasted_iota(jnp.int32, (CHUNK, CHUNK), 0)
    c_i = lax.broadcasted_iota(jnp.int32, (CHUNK, CHUNK), 1)
    tri = (r_i >= c_i).astype(jnp.float32)
    carry = jnp.zeros((1, E), jnp.float32)
    psel_parts = []
    for k in range(S // CHUNK):
        blk = oh[k * CHUNK:(k + 1) * CHUNK, :]
        pr = lax.dot_general(
            tri, blk, (((1,), (0,)), ((), ())),
            preferred_element_type=jnp.float32) + carry
        carry = pr[CHUNK - 1:CHUNK, :]
        psel_parts.append(jnp.sum(pr * blk, axis=1, keepdims=True))
    psel = jnp.concatenate(psel_parts, axis=0)

    tok = lax.broadcasted_iota(jnp.int32, (S, 1), 0)
    routed = psel <= jnp.float32(CAP)
    slot = am * CAP + psel.astype(jnp.int32) - 1
    src_ref[...] = jnp.where(routed, S + slot, tok)


_router_call = pl.pallas_call(
    _router_body,
    out_shape=(
        jax.ShapeDtypeStruct((S, E), jnp.float32),
        jax.ShapeDtypeStruct((S, 1), jnp.int32),
        jax.ShapeDtypeStruct((S, 1), jnp.float32),
    ),
)


def _sc_dispatch_body(src_hbm, x_hbm, xg_hbm,
                      src_v, idx_v, idxc_v, rows_v, sem1, sem3):
    wid = lax.axis_index("s") * 2 + lax.axis_index("c")
    base = wid * ROWS_PER_W

    cp_src = pltpu.async_copy(src_hbm, src_v, sem1)

    def init_body(i, _):
        idx_v[pl.ds(base + i * LANES, LANES)] = jnp.zeros((LANES,), jnp.int32)
        return _

    lax.fori_loop(0, ROWS_PER_W // LANES, init_body, 0, unroll=4)

    cp_src.wait()

    def scatter_body(i, _):
        v = src_v[pl.ds(i * LANES, LANES)]
        slot = v - S
        m = slot >= 0
        toks = lax.iota(jnp.int32, LANES) + i * LANES
        plsc.store_scatter(idx_v, [slot], toks, mask=m)
        return _

    lax.fori_loop(0, S // LANES, scatter_body, 0, unroll=4)

    def copy_body(i, _):
        idxc_v[pl.ds(i * LANES, LANES)] = idx_v[pl.ds(base + i * LANES, LANES)]
        return _

    lax.fori_loop(0, ROWS_PER_W // LANES, copy_body, 0, unroll=4)

    gather = pltpu.async_copy(x_hbm.at[idxc_v], rows_v, sem3)
    gather.wait()
    pltpu.sync_copy(rows_v, xg_hbm.at[pl.ds(base, ROWS_PER_W)])


@functools.cache
def _sc_dispatch_call():
    return functools.partial(
        pl.kernel,
        mesh=plsc.VectorSubcoreMesh(core_axis_name="c", subcore_axis_name="s"),
        compiler_params=pltpu.CompilerParams(needs_layout_passes=False),
        out_type=jax.ShapeDtypeStruct((S, D), jnp.float32),
        scratch_types=[
            pltpu.VMEM((S,), jnp.int32),
            pltpu.VMEM((S,), jnp.int32),
            pltpu.VMEM((ROWS_PER_W,), jnp.int32),
            pltpu.VMEM((ROWS_PER_W, D), jnp.float32),
            pltpu.SemaphoreType.DMA,
            pltpu.SemaphoreType.DMA,
        ],
    )(_sc_dispatch_body)


def _expert_body(xg_ref, x_ref, src_ref, rp_ref, wi_ref, wo_ref, y_ref):
    i = pl.program_id(0)

    @pl.when(i == 0)
    def _base():
        routed = src_ref[...] >= S
        y_ref[...] = jnp.where(routed, 0.0, rp_ref[...] * x_ref[...])

    @pl.when(i > 0)
    def _expert():
        xg = xg_ref[...]
        wi = wi_ref[0]
        wo = wo_ref[0]
        h = lax.dot_general(
            xg, wi, (((1,), (1,)), ((), ())),
            preferred_element_type=jnp.float32)
        h = 0.5 * h * (1.0 + lax.erf(h * 0.7071067811865476))
        out = lax.dot_general(
            h, wo, (((1,), (1,)), ((), ())),
            preferred_element_type=jnp.float32)
        c = lax.broadcasted_iota(jnp.int32, (S, CAP), 1)
        de = (src_ref[...] == (S + (i - 1) * CAP + c)).astype(jnp.float32)
        rps = lax.dot_general(
            de, rp_ref[...], (((0,), (0,)), ((), ())),
            preferred_element_type=jnp.float32)
        y_ref[...] += lax.dot_general(
            de, out * rps, (((1,), (0,)), ((), ())),
            preferred_element_type=jnp.float32)


_expert_call = pl.pallas_call(
    _expert_body,
    grid=(E + 1,),
    in_specs=[
        pl.BlockSpec((CAP, D), lambda i: (jnp.maximum(i - 1, 0), 0)),
        pl.BlockSpec((S, D), lambda i: (0, 0)),
        pl.BlockSpec((S, 1), lambda i: (0, 0)),
        pl.BlockSpec((S, 1), lambda i: (0, 0)),
        pl.BlockSpec((1, FFN, D), lambda i: (jnp.maximum(i - 1, 0), 0, 0)),
        pl.BlockSpec((1, D, FFN), lambda i: (jnp.maximum(i - 1, 0), 0, 0)),
    ],
    out_specs=pl.BlockSpec((S, D), lambda i: (0, 0)),
    out_shape=jax.ShapeDtypeStruct((S, D), jnp.float32),
)


def kernel(x, Wr, Wi, Wo):
    x2 = x.reshape(S, D)
    logits, src, rp = _router_call(x2, Wr)
    xg = _sc_dispatch_call()(src.reshape(S), x2)
    y2 = _expert_call(xg, x2, src, rp, Wi, Wo)
    return y2.reshape(1, S, D), logits.reshape(1, S, E)
